# baseline (speedup 1.0000x reference)
import functools

import jax
import jax.numpy as jnp
from jax import lax
from jax.experimental import pallas as pl
from jax.experimental.pallas import tpu as pltpu
from jax.experimental.pallas import tpu_sc as plsc

FEAT = 128
NRBF = 20
CUTOFF = 5.0
N = 10000
E = 160000
H4 = 4 * FEAT
TW = H4 + 6 * FEAT
PW = 7 * FEAT
NPL = 13

NC = 2
NS = 16
NT = NC * NS
EPT = E // NT
GCH = 40
NGC = EPT // GCH

RPB = 128
NROW = E // RPB
RPC = NROW // NC
RPT = RPC // NS
ZR = 208

_HI = lax.Precision.HIGHEST


def _table_body(s_ref, vx, vy, vz, wx, wy, wz, W1_ref, b1_ref, W2_ref, b2_ref,
                out_ref):
    x = s_ref[...]
    pre = jnp.dot(x, W1_ref[...], preferred_element_type=jnp.float32,
                  precision=_HI) + b1_ref[...]
    act = pre * jax.nn.sigmoid(pre)
    h = jnp.dot(act, W2_ref[...], preferred_element_type=jnp.float32,
                precision=_HI) + b2_ref[...]
    out_ref[...] = jnp.concatenate(
        [h, vx[...], vy[...], vz[...], wx[...], wy[...], wz[...]], axis=1)


def _build_table(s_j, vx, vy, vz, wx, wy, wz, W1, b1, W2, b2):
    nb = 10
    bn = N // nb
    row = lambda i: (i, 0)
    full = lambda shape: pl.BlockSpec(shape, lambda i: (0, 0))
    return pl.pallas_call(
        _table_body,
        grid=(nb,),
        in_specs=[pl.BlockSpec((bn, FEAT), row)] * 7 + [
            full((FEAT, FEAT)), full((1, FEAT)),
            full((FEAT, H4)), full((1, H4))],
        out_specs=pl.BlockSpec((bn, TW), row),
        out_shape=jax.ShapeDtypeStruct((N, TW), jnp.float32),
    )(s_j, vx, vy, vz, wx, wy, wz, W1, b1.reshape(1, FEAT), W2,
      b2.reshape(1, H4))


def _gather_rows(table, i1r, e_g, ngc):
    mesh = plsc.VectorSubcoreMesh(core_axis_name="c", subcore_axis_name="s")
    ept = ngc * GCH

    @functools.partial(
        pl.kernel,
        out_type=jax.ShapeDtypeStruct((e_g, TW), jnp.float32),
        mesh=mesh,
        scratch_types=[
            pltpu.VMEM((ngc, GCH), jnp.int32),
            pltpu.VMEM((GCH, TW), jnp.float32),
            pltpu.VMEM((GCH, TW), jnp.float32),
            pltpu.SemaphoreType.DMA,
            pltpu.SemaphoreType.DMA,
        ],
    )
    def k(t_hbm, idx_hbm, out_hbm, idx_v, buf_a, buf_b, sem_a, sem_b):
        w = lax.axis_index("c") * NS + lax.axis_index("s")
        pltpu.sync_copy(idx_hbm.at[w], idx_v)
        base = w * ept
        out_at = lambda kk: out_hbm.at[pl.ds(base + kk * GCH, GCH)]

        pltpu.async_copy(t_hbm.at[idx_v.at[0]], buf_a, sem_a)

        @pl.loop(0, ngc // 2)
        def _(i):
            kk = 2 * i
            pltpu.async_copy(t_hbm.at[idx_v.at[kk + 1]], buf_b, sem_b)
            pltpu.make_async_copy(t_hbm.at[idx_v.at[kk]], buf_a, sem_a).wait()
            pltpu.sync_copy(buf_a, out_at(kk))

            @pl.when(kk + 2 < ngc)
            def _():
                pltpu.async_copy(t_hbm.at[idx_v.at[kk + 2]], buf_a, sem_a)

            pltpu.make_async_copy(
                t_hbm.at[idx_v.at[kk + 1]], buf_b, sem_b).wait()
            pltpu.sync_copy(buf_b, out_at(kk + 1))

        if ngc % 2:
            pltpu.make_async_copy(
                t_hbm.at[idx_v.at[ngc - 1]], buf_a, sem_a).wait()
            pltpu.sync_copy(buf_a, out_at(ngc - 1))

    return k(table, i1r)


def _rchain_body(rx_ref, ry_ref, rz_ref, out_ref):
    rx = rx_ref[...]
    ry = ry_ref[...]
    rz = rz_ref[...]
    d2 = rx * rx + ry * ry + rz * rz + 3e-8
    inv = lax.rsqrt(d2)
    dist = d2 * inv
    a = (jnp.pi / CUTOFF) * dist
    s1 = jnp.sin(a)
    c1 = jnp.cos(a)
    env = jnp.where(dist < CUTOFF, 0.5 * (c1 + 1.0), 0.0)
    m = env * inv
    two_c = 2.0 * c1
    rows = [m * s1]
    sk_prev, sk = s1, two_c * s1
    rows.append(m * sk)
    for _ in range(2, NRBF):
        sk_prev, sk = sk, two_c * sk - sk_prev
        rows.append(m * sk)
    rows.append(env)
    rows.append(rx * inv)
    rows.append(ry * inv)
    rows.append(rz * inv)
    out_ref[...] = jnp.concatenate(rows, axis=0)


def _rchain(rx, ry, rz):
    eb = 3200
    ng = E // eb
    col = lambda i: (0, i)
    return pl.pallas_call(
        _rchain_body,
        grid=(ng,),
        in_specs=[pl.BlockSpec((1, eb), col)] * 3,
        out_specs=pl.BlockSpec((NRBF + 4, eb), col),
        out_shape=jax.ShapeDtypeStruct((NRBF + 4, E), jnp.float32),
    )(rx, ry, rz)


def _edge_body(g_ref, rb_ref, Wd_ref, p_ref):
    rb = rb_ref[...]
    ws = jnp.dot(rb[:, 0:NRBF + 1], Wd_ref[...],
                 preferred_element_type=jnp.float32, precision=_HI)
    ux = rb[:, NRBF + 1:NRBF + 2]
    uy = rb[:, NRBF + 2:NRBF + 3]
    uz = rb[:, NRBF + 3:NRBF + 4]
    g = g_ref[...]
    t0 = g[:, 0:FEAT] * ws[:, 0:FEAT]
    t1 = g[:, FEAT:2 * FEAT] * ws[:, FEAT:2 * FEAT]
    t2 = g[:, 2 * FEAT:3 * FEAT] * ws[:, 2 * FEAT:3 * FEAT]
    t3 = g[:, 3 * FEAT:4 * FEAT] * ws[:, 3 * FEAT:4 * FEAT]
    v1x = g[:, H4:H4 + FEAT]
    v1y = g[:, H4 + FEAT:H4 + 2 * FEAT]
    v1z = g[:, H4 + 2 * FEAT:H4 + 3 * FEAT]
    w1x = g[:, H4 + 3 * FEAT:H4 + 4 * FEAT]
    w1y = g[:, H4 + 4 * FEAT:H4 + 5 * FEAT]
    w1z = g[:, H4 + 5 * FEAT:H4 + 6 * FEAT]
    p_ref[...] = jnp.concatenate([
        t1,
        t2 * ux + t0 * v1x,
        t2 * uy + t0 * v1y,
        t2 * uz + t0 * v1z,
        t3 * w1x,
        t3 * w1y,
        t3 * w1z,
    ], axis=1)


def _edge_stage(G, RB, Wd21, e_g):
    eb = 640
    ng = e_g // eb
    row = lambda i: (i, 0)
    return pl.pallas_call(
        _edge_body,
        grid=(ng,),
        in_specs=[
            pl.BlockSpec((eb, TW), row),
            pl.BlockSpec((eb, NRBF + 4), row),
            pl.BlockSpec((NRBF + 1, H4), lambda i: (0, 0)),
        ],
        out_specs=pl.BlockSpec((eb, PW), row),
        out_shape=jax.ShapeDtypeStruct((e_g, PW), jnp.float32),
    )(G, RB, Wd21)


def _scatter_planes(P, G, i0r, row_base, nrows, init):
    mesh = plsc.VectorSubcoreMesh(core_axis_name="c", subcore_axis_name="s")
    psc = nrows // NC
    pt = psc // NS
    extra = psc - pt * NS
    slab = pt + extra

    def body(p_hbm, g_hbm, idx_hbm, out_hbm, idx_v, pay_a, pay_b, acc,
             sem_a, sem_b, init_hbm):
        c = lax.axis_index("c")
        w = lax.axis_index("s")

        start = row_base + c * psc + w * pt
        pltpu.sync_copy(idx_hbm.at[pl.ds(start, slab)], idx_v)

        if init is None:
            @pl.loop(0, RPB)
            def _(i):
                @pl.loop(0, FEAT // 16)
                def _(l):
                    pay_a.at[i, pl.ds(l * 16, 16)][...] = jnp.zeros(
                        (16,), jnp.float32)

        for p in range(NPL):
            if init is None:
                for z in range(4):
                    pltpu.sync_copy(pay_a,
                                    acc.at[pl.ds(w * 624 + z * RPB, RPB)])
                pltpu.sync_copy(pay_a.at[pl.ds(0, 112)],
                                acc.at[pl.ds(w * 624 + 4 * RPB, 112)])

                @pl.when(w == NS - 1)
                def _():
                    pltpu.sync_copy(pay_a.at[pl.ds(0, 16)],
                                    acc.at[pl.ds(N - 16, 16)])
            else:
                pltpu.sync_copy(init_hbm.at[c, p, pl.ds(w * 624, 624)],
                                acc.at[pl.ds(w * 624, 624)])

                @pl.when(w == NS - 1)
                def _():
                    pltpu.sync_copy(init_hbm.at[c, p, pl.ds(N - 16, 16)],
                                    acc.at[pl.ds(N - 16, 16)])

            plsc.subcore_barrier()

            if p < 7:
                src, col = p_hbm, p * FEAT
            else:
                src, col = g_hbm, H4 + (p - 7) * FEAT

            def src_at(j):
                return src.at[pl.ds((start - row_base + j) * RPB, RPB),
                              pl.ds(col, FEAT)]

            def fetch(j, buf, sem):
                pltpu.async_copy(src_at(j), buf, sem)

            def drain_add(j, buf, sem):
                pltpu.make_async_copy(src_at(j), buf, sem).wait()
                pltpu.sync_copy(buf, acc.at[idx_v.at[j, 0]], add=True)

            fetch(0, pay_b, sem_b)

            @pl.loop(0, pt // 2)
            def _(i):
                j = 2 * i
                fetch(j + 1, pay_a, sem_a)
                drain_add(j, pay_b, sem_b)

                @pl.when(j + 2 < pt)
                def _():
                    fetch(j + 2, pay_b, sem_b)

                drain_add(j + 1, pay_a, sem_a)

            if pt % 2:
                drain_add(pt - 1, pay_b, sem_b)

            @pl.when(w == NS - 1)
            def _():
                @pl.loop(pt, pt + extra)
                def _(j):
                    pltpu.sync_copy(src_at(j), pay_b)
                    pltpu.sync_copy(pay_b, acc.at[idx_v.at[j, 0]], add=True)

            plsc.subcore_barrier()
            pltpu.sync_copy(acc.at[pl.ds(w * 624, 624)],
                            out_hbm.at[c, p, pl.ds(w * 624, 624)])

            @pl.when(w == NS - 1)
            def _():
                pltpu.sync_copy(acc.at[pl.ds(N - 16, 16)],
                                out_hbm.at[c, p, pl.ds(N - 16, 16)])

            if init is None and p + 1 < NPL:
                @pl.loop(0, RPB)
                def _(i):
                    @pl.loop(0, FEAT // 16)
                    def _(l):
                        pay_a.at[i, pl.ds(l * 16, 16)][...] = jnp.zeros(
                            (16,), jnp.float32)

    args = [P, G, i0r]
    if init is None:
        def k_no_init(p_hbm, g_hbm, idx_hbm, out_hbm, idx_v, pay_a, pay_b,
                      acc, sem_a, sem_b):
            body(p_hbm, g_hbm, idx_hbm, out_hbm, idx_v, pay_a, pay_b, acc,
                 sem_a, sem_b, None)
        fn = k_no_init
    else:
        def k_init(p_hbm, g_hbm, idx_hbm, init_hbm, out_hbm, idx_v, pay_a,
                   pay_b, acc, sem_a, sem_b):
            body(p_hbm, g_hbm, idx_hbm, out_hbm, idx_v, pay_a, pay_b, acc,
                 sem_a, sem_b, init_hbm)
        fn = k_init
        args.append(init)

    k = functools.partial(
        pl.kernel,
        out_type=jax.ShapeDtypeStruct((NC, NPL, N, FEAT), jnp.float32),
        mesh=mesh,
        scratch_types=[
            pltpu.VMEM((slab, 1, RPB), jnp.int32),
            pltpu.VMEM((RPB, FEAT), jnp.float32),
            pltpu.VMEM((RPB, FEAT), jnp.float32),
            pltpu.VMEM_SHARED((N, FEAT), jnp.float32),
            pltpu.SemaphoreType.DMA,
            pltpu.SemaphoreType.DMA,
        ],
    )(fn)
    return k(*args)


def _node_body(a_ref, s_ref, sb_ref, vx_r, vy_r, vz_r, wx_r, wy_r, wz_r,
               Wvb_ref, Wv_ref,
               dh_ref, dhb_ref, dvx_r, dvy_r, dvz_r, dbx_r, dby_r, dbz_r):
    agg = lambda p: a_ref[p] + a_ref[NPL + p]
    dh = agg(0)
    pv = (agg(1), agg(2), agg(3))
    B = (agg(4), agg(5), agg(6))
    Av = (agg(7), agg(8), agg(9))
    Aw = (agg(10), agg(11), agg(12))
    s = s_ref[...]
    sb = sb_ref[...]
    v = (vx_r[...], vy_r[...], vz_r[...])
    vb = (wx_r[...], wy_r[...], wz_r[...])
    dh_ref[...] = dh
    dhb_ref[...] = v[0] * B[0] + v[1] * B[1] + v[2] * B[2]

    def cross(a, b, d):
        i, j = (d + 1) % 3, (d + 2) % 3
        return a[i] * b[j] - a[j] * b[i]

    Wvb = Wvb_ref[...]
    Wv = Wv_ref[...]
    douts = (dvx_r, dvy_r, dvz_r)
    bouts = (dbx_r, dby_r, dbz_r)
    for d in range(3):
        catb = jnp.concatenate(
            [s * Aw[d], sb * Av[d], cross(v, Av, d), cross(vb, Aw, d)], axis=1)
        bouts[d][...] = jnp.dot(catb, Wvb, preferred_element_type=jnp.float32,
                                precision=_HI)
        catv = jnp.concatenate(
            [s * Av[d], sb * Aw[d], cross(v, Aw, d)], axis=1)
        douts[d][...] = pv[d] + jnp.dot(
            catv, Wv, preferred_element_type=jnp.float32, precision=_HI)


def _node_stage(A, s_j, sbar_j, vx, vy, vz, wx, wy, wz, Wdvbar, Wdv):
    nb = 10
    bn = N // nb
    row = lambda i: (i, 0)
    o = pl.BlockSpec((bn, FEAT), row)
    outs = [jax.ShapeDtypeStruct((N, FEAT), jnp.float32)] * 8
    return pl.pallas_call(
        _node_body,
        grid=(nb,),
        in_specs=[
            pl.BlockSpec((2 * NPL, bn, FEAT), lambda i: (0, i, 0)),
        ] + [pl.BlockSpec((bn, FEAT), row)] * 8 + [
            pl.BlockSpec((H4, FEAT), lambda i: (0, 0)),
            pl.BlockSpec((3 * FEAT, FEAT), lambda i: (0, 0)),
        ],
        out_specs=[o] * 8,
        out_shape=outs,
    )(A, s_j, sbar_j, vx, vy, vz, wx, wy, wz, Wdvbar, Wdv)


def kernel(s_j, sbar_j, v_j, vbar_j, r_ij, nbrs, W1, b1, W2, b2, Wd, bd,
           Wdvbar, Wdv):
    i0 = nbrs[:, 0].astype(jnp.int32)
    i1 = nbrs[:, 1].astype(jnp.int32)
    vx, vy, vz = v_j[:, :, 0], v_j[:, :, 1], v_j[:, :, 2]
    wx, wy, wz = vbar_j[:, :, 0], vbar_j[:, :, 1], vbar_j[:, :, 2]
    rx, ry, rz = (r_ij[:, 0].reshape(1, E), r_ij[:, 1].reshape(1, E),
                  r_ij[:, 2].reshape(1, E))
    Wd21 = jnp.concatenate([Wd, bd.reshape(1, H4)], axis=0)

    table = _build_table(s_j, vx, vy, vz, wx, wy, wz, W1, b1, W2, b2)
    RB = _rchain(rx, ry, rz).T
    i0r = i0.reshape(NROW, 1, RPB)

    ea = 62 * NT * GCH
    eb_ = E - ea
    ga = _gather_rows(table, i1[:ea].reshape(NT, 62, GCH), ea, 62)
    gb = _gather_rows(table, i1[ea:].reshape(NT, 63, GCH), eb_, 63)
    pa = _edge_stage(ga, RB[:ea], Wd21, ea)
    pb = _edge_stage(gb, RB[ea:], Wd21, eb_)
    parts_a = _scatter_planes(pa, ga, i0r, 0, ea // RPB, None)
    parts = _scatter_planes(pb, gb, i0r, ea // RPB, eb_ // RPB, parts_a)
    A = parts.reshape(NC * NPL, N, FEAT)
    dh, dhbar, dvx, dvy, dvz, dbx, dby, dbz = _node_stage(
        A, s_j, sbar_j, vx, vy, vz, wx, wy, wz, Wdvbar, Wdv)
    dv = jnp.stack([dvx, dvy, dvz], axis=-1)
    dvbar = jnp.stack([dbx, dby, dbz], axis=-1)
    return dh, dhbar, dv, dvbar


# speedup vs baseline: 1.1499x; 1.1499x over previous
import functools

import jax
import jax.numpy as jnp
from jax import lax
from jax.experimental import pallas as pl
from jax.experimental.pallas import tpu as pltpu
from jax.experimental.pallas import tpu_sc as plsc

FEAT = 128
NRBF = 20
CUTOFF = 5.0
N = 10000
E = 160000
H4 = 4 * FEAT
TW = H4 + 6 * FEAT
PW = 7 * FEAT
NPL = 13

NC = 2
NS = 16
NT = NC * NS
EPT = E // NT
GCH = 40
NGC = EPT // GCH

RPB = 128
NROW = E // RPB
RPC = NROW // NC
RPT = RPC // NS
ZR = 208

_HI = lax.Precision.HIGHEST


def _table_body(s_ref, vx, vy, vz, wx, wy, wz, W1_ref, b1_ref, W2_ref, b2_ref,
                out_ref):
    x = s_ref[...]
    pre = jnp.dot(x, W1_ref[...], preferred_element_type=jnp.float32,
                  precision=_HI) + b1_ref[...]
    act = pre * jax.nn.sigmoid(pre)
    h = jnp.dot(act, W2_ref[...], preferred_element_type=jnp.float32,
                precision=_HI) + b2_ref[...]
    out_ref[...] = jnp.concatenate(
        [h, vx[...], vy[...], vz[...], wx[...], wy[...], wz[...]], axis=1)


def _build_table(s_j, vx, vy, vz, wx, wy, wz, W1, b1, W2, b2):
    nb = 10
    bn = N // nb
    row = lambda i: (i, 0)
    full = lambda shape: pl.BlockSpec(shape, lambda i: (0, 0))
    return pl.pallas_call(
        _table_body,
        grid=(nb,),
        in_specs=[pl.BlockSpec((bn, FEAT), row)] * 7 + [
            full((FEAT, FEAT)), full((1, FEAT)),
            full((FEAT, H4)), full((1, H4))],
        out_specs=pl.BlockSpec((bn, TW), row),
        out_shape=jax.ShapeDtypeStruct((N, TW), jnp.float32),
    )(s_j, vx, vy, vz, wx, wy, wz, W1, b1.reshape(1, FEAT), W2,
      b2.reshape(1, H4))


def _gather_rows(table, i1r, e_g):
    mesh = plsc.VectorSubcoreMesh(core_axis_name="c", subcore_axis_name="s")
    nchunks = e_g // GCH
    small = nchunks // NT
    nbig = nchunks - small * NT
    ngc = i1r.shape[1]
    assert ngc == small + (1 if nbig else 0)

    @functools.partial(
        pl.kernel,
        out_type=jax.ShapeDtypeStruct((e_g, TW), jnp.float32),
        mesh=mesh,
        scratch_types=[
            pltpu.VMEM((ngc, GCH), jnp.int32),
            pltpu.VMEM((GCH, TW), jnp.float32),
            pltpu.VMEM((GCH, TW), jnp.float32),
            pltpu.SemaphoreType.DMA,
            pltpu.SemaphoreType.DMA,
        ],
    )
    def k(t_hbm, idx_hbm, out_hbm, idx_v, buf_a, buf_b, sem_a, sem_b):
        w = lax.axis_index("c") * NS + lax.axis_index("s")
        pltpu.sync_copy(idx_hbm.at[w], idx_v)
        base = w * (small * GCH) + jnp.minimum(w, nbig) * GCH
        out_at = lambda kk: out_hbm.at[pl.ds(base + kk * GCH, GCH)]

        pltpu.async_copy(t_hbm.at[idx_v.at[0]], buf_a, sem_a)

        @pl.loop(0, small // 2)
        def _(i):
            kk = 2 * i
            pltpu.async_copy(t_hbm.at[idx_v.at[kk + 1]], buf_b, sem_b)
            pltpu.make_async_copy(t_hbm.at[idx_v.at[kk]], buf_a, sem_a).wait()
            pltpu.sync_copy(buf_a, out_at(kk))

            @pl.when(kk + 2 < small)
            def _():
                pltpu.async_copy(t_hbm.at[idx_v.at[kk + 2]], buf_a, sem_a)

            pltpu.make_async_copy(
                t_hbm.at[idx_v.at[kk + 1]], buf_b, sem_b).wait()
            pltpu.sync_copy(buf_b, out_at(kk + 1))

        if small % 2:
            pltpu.make_async_copy(
                t_hbm.at[idx_v.at[small - 1]], buf_a, sem_a).wait()
            pltpu.sync_copy(buf_a, out_at(small - 1))

        if nbig:
            @pl.when(w < nbig)
            def _():
                pltpu.async_copy(
                    t_hbm.at[idx_v.at[small]], buf_b, sem_b).wait()
                pltpu.sync_copy(buf_b, out_at(small))

    return k(table, i1r)


def _rchain_body(rx_ref, ry_ref, rz_ref, out_ref):
    rx = rx_ref[...]
    ry = ry_ref[...]
    rz = rz_ref[...]
    d2 = rx * rx + ry * ry + rz * rz + 3e-8
    inv = lax.rsqrt(d2)
    dist = d2 * inv
    a = (jnp.pi / CUTOFF) * dist
    s1 = jnp.sin(a)
    c1 = jnp.cos(a)
    env = jnp.where(dist < CUTOFF, 0.5 * (c1 + 1.0), 0.0)
    m = env * inv
    two_c = 2.0 * c1
    rows = [m * s1]
    sk_prev, sk = s1, two_c * s1
    rows.append(m * sk)
    for _ in range(2, NRBF):
        sk_prev, sk = sk, two_c * sk - sk_prev
        rows.append(m * sk)
    rows.append(env)
    rows.append(rx * inv)
    rows.append(ry * inv)
    rows.append(rz * inv)
    out_ref[...] = jnp.concatenate(rows, axis=0)


def _rchain(rx, ry, rz):
    eb = 3200
    ng = E // eb
    col = lambda i: (0, i)
    return pl.pallas_call(
        _rchain_body,
        grid=(ng,),
        in_specs=[pl.BlockSpec((1, eb), col)] * 3,
        out_specs=pl.BlockSpec((NRBF + 4, eb), col),
        out_shape=jax.ShapeDtypeStruct((NRBF + 4, E), jnp.float32),
    )(rx, ry, rz)


def _edge_body(g_ref, rb_ref, Wd_ref, p_ref):
    rb = rb_ref[...]
    ws = jnp.dot(rb[:, 0:NRBF + 1], Wd_ref[...],
                 preferred_element_type=jnp.float32, precision=_HI)
    ux = rb[:, NRBF + 1:NRBF + 2]
    uy = rb[:, NRBF + 2:NRBF + 3]
    uz = rb[:, NRBF + 3:NRBF + 4]
    g = g_ref[...]
    t0 = g[:, 0:FEAT] * ws[:, 0:FEAT]
    t1 = g[:, FEAT:2 * FEAT] * ws[:, FEAT:2 * FEAT]
    t2 = g[:, 2 * FEAT:3 * FEAT] * ws[:, 2 * FEAT:3 * FEAT]
    t3 = g[:, 3 * FEAT:4 * FEAT] * ws[:, 3 * FEAT:4 * FEAT]
    v1x = g[:, H4:H4 + FEAT]
    v1y = g[:, H4 + FEAT:H4 + 2 * FEAT]
    v1z = g[:, H4 + 2 * FEAT:H4 + 3 * FEAT]
    w1x = g[:, H4 + 3 * FEAT:H4 + 4 * FEAT]
    w1y = g[:, H4 + 4 * FEAT:H4 + 5 * FEAT]
    w1z = g[:, H4 + 5 * FEAT:H4 + 6 * FEAT]
    p_ref[...] = jnp.concatenate([
        t1,
        t2 * ux + t0 * v1x,
        t2 * uy + t0 * v1y,
        t2 * uz + t0 * v1z,
        t3 * w1x,
        t3 * w1y,
        t3 * w1z,
    ], axis=1)


def _edge_stage(G, RB, Wd21, e_g):
    eb = 640
    ng = e_g // eb
    row = lambda i: (i, 0)
    return pl.pallas_call(
        _edge_body,
        grid=(ng,),
        in_specs=[
            pl.BlockSpec((eb, TW), row),
            pl.BlockSpec((eb, NRBF + 4), row),
            pl.BlockSpec((NRBF + 1, H4), lambda i: (0, 0)),
        ],
        out_specs=pl.BlockSpec((eb, PW), row),
        out_shape=jax.ShapeDtypeStruct((e_g, PW), jnp.float32),
    )(G, RB, Wd21)


def _scatter_planes(PA, GA, PB, GB, i0r):
    mesh = plsc.VectorSubcoreMesh(core_axis_name="c", subcore_axis_name="s")

    @functools.partial(
        pl.kernel,
        out_type=jax.ShapeDtypeStruct((NC, NPL, N, FEAT), jnp.float32),
        mesh=mesh,
        scratch_types=[
            pltpu.VMEM((RPT + 1, 1, RPB), jnp.int32),
            pltpu.VMEM((RPB, FEAT), jnp.float32),
            pltpu.VMEM((RPB, FEAT), jnp.float32),
            pltpu.VMEM_SHARED((N, FEAT), jnp.float32),
            pltpu.SemaphoreType.DMA,
            pltpu.SemaphoreType.DMA,
        ],
    )
    def k(pa_hbm, ga_hbm, pb_hbm, gb_hbm, idx_hbm, out_hbm,
          idx_v, pay_a, pay_b, acc, sem_a, sem_b):
        c = lax.axis_index("c")
        w = lax.axis_index("s")

        start = c * RPC + w * RPT
        pltpu.sync_copy(idx_hbm.at[pl.ds(start, RPT + 1)], idx_v)

        def zero_fill():
            @pl.loop(0, RPB)
            def _(i):
                @pl.loop(0, FEAT // 16)
                def _(l):
                    pay_a.at[i, pl.ds(l * 16, 16)][...] = jnp.zeros(
                        (16,), jnp.float32)

        zero_fill()

        for p in range(NPL):
            for z in range(4):
                pltpu.sync_copy(pay_a,
                                acc.at[pl.ds(w * 624 + z * RPB, RPB)])
            pltpu.sync_copy(pay_a.at[pl.ds(0, 112)],
                            acc.at[pl.ds(w * 624 + 4 * RPB, 112)])

            @pl.when(w == NS - 1)
            def _():
                pltpu.sync_copy(pay_a.at[pl.ds(0, 16)],
                                acc.at[pl.ds(N - 16, 16)])

            plsc.subcore_barrier()

            def emit(p_src, g_src):
                if p < 7:
                    src, col = p_src, p * FEAT
                else:
                    src, col = g_src, H4 + (p - 7) * FEAT

                def src_at(j):
                    return src.at[pl.ds((w * RPT + j) * RPB, RPB),
                                  pl.ds(col, FEAT)]

                def fetch(j, buf, sem):
                    pltpu.async_copy(src_at(j), buf, sem)

                def drain_add(j, buf, sem):
                    pltpu.make_async_copy(src_at(j), buf, sem).wait()
                    pltpu.sync_copy(buf, acc.at[idx_v.at[j, 0]], add=True)

                fetch(0, pay_b, sem_b)

                @pl.loop(0, RPT // 2)
                def _(i):
                    j = 2 * i
                    fetch(j + 1, pay_a, sem_a)
                    drain_add(j, pay_b, sem_b)

                    @pl.when(j + 2 < RPT)
                    def _():
                        fetch(j + 2, pay_b, sem_b)

                    drain_add(j + 1, pay_a, sem_a)

                if RPT % 2:
                    drain_add(RPT - 1, pay_b, sem_b)

                @pl.when(w == NS - 1)
                def _():
                    pltpu.sync_copy(src_at(RPT), pay_b)
                    pltpu.sync_copy(pay_b, acc.at[idx_v.at[RPT, 0]],
                                    add=True)

            @pl.when(c == 0)
            def _():
                emit(pa_hbm, ga_hbm)

            @pl.when(c == 1)
            def _():
                emit(pb_hbm, gb_hbm)

            plsc.subcore_barrier()
            pltpu.sync_copy(acc.at[pl.ds(w * 624, 624)],
                            out_hbm.at[c, p, pl.ds(w * 624, 624)])

            @pl.when(w == NS - 1)
            def _():
                pltpu.sync_copy(acc.at[pl.ds(N - 16, 16)],
                                out_hbm.at[c, p, pl.ds(N - 16, 16)])

            if p + 1 < NPL:
                zero_fill()

    return k(PA, GA, PB, GB, i0r)


def _node_body(a_ref, s_ref, sb_ref, vx_r, vy_r, vz_r, wx_r, wy_r, wz_r,
               Wvb_ref, Wv_ref,
               dh_ref, dhb_ref, dvx_r, dvy_r, dvz_r, dbx_r, dby_r, dbz_r):
    agg = lambda p: a_ref[p] + a_ref[NPL + p]
    dh = agg(0)
    pv = (agg(1), agg(2), agg(3))
    B = (agg(4), agg(5), agg(6))
    Av = (agg(7), agg(8), agg(9))
    Aw = (agg(10), agg(11), agg(12))
    s = s_ref[...]
    sb = sb_ref[...]
    v = (vx_r[...], vy_r[...], vz_r[...])
    vb = (wx_r[...], wy_r[...], wz_r[...])
    dh_ref[...] = dh
    dhb_ref[...] = v[0] * B[0] + v[1] * B[1] + v[2] * B[2]

    def cross(a, b, d):
        i, j = (d + 1) % 3, (d + 2) % 3
        return a[i] * b[j] - a[j] * b[i]

    Wvb = Wvb_ref[...]
    Wv = Wv_ref[...]
    douts = (dvx_r, dvy_r, dvz_r)
    bouts = (dbx_r, dby_r, dbz_r)
    for d in range(3):
        catb = jnp.concatenate(
            [s * Aw[d], sb * Av[d], cross(v, Av, d), cross(vb, Aw, d)], axis=1)
        bouts[d][...] = jnp.dot(catb, Wvb, preferred_element_type=jnp.float32,
                                precision=_HI)
        catv = jnp.concatenate(
            [s * Av[d], sb * Aw[d], cross(v, Aw, d)], axis=1)
        douts[d][...] = pv[d] + jnp.dot(
            catv, Wv, preferred_element_type=jnp.float32, precision=_HI)


def _node_stage(A, s_j, sbar_j, vx, vy, vz, wx, wy, wz, Wdvbar, Wdv):
    nb = 10
    bn = N // nb
    row = lambda i: (i, 0)
    o = pl.BlockSpec((bn, FEAT), row)
    outs = [jax.ShapeDtypeStruct((N, FEAT), jnp.float32)] * 8
    return pl.pallas_call(
        _node_body,
        grid=(nb,),
        in_specs=[
            pl.BlockSpec((2 * NPL, bn, FEAT), lambda i: (0, i, 0)),
        ] + [pl.BlockSpec((bn, FEAT), row)] * 8 + [
            pl.BlockSpec((H4, FEAT), lambda i: (0, 0)),
            pl.BlockSpec((3 * FEAT, FEAT), lambda i: (0, 0)),
        ],
        out_specs=[o] * 8,
        out_shape=outs,
    )(A, s_j, sbar_j, vx, vy, vz, wx, wy, wz, Wdvbar, Wdv)


def kernel(s_j, sbar_j, v_j, vbar_j, r_ij, nbrs, W1, b1, W2, b2, Wd, bd,
           Wdvbar, Wdv):
    i0 = nbrs[:, 0].astype(jnp.int32)
    i1 = nbrs[:, 1].astype(jnp.int32)
    vx, vy, vz = v_j[:, :, 0], v_j[:, :, 1], v_j[:, :, 2]
    wx, wy, wz = vbar_j[:, :, 0], vbar_j[:, :, 1], vbar_j[:, :, 2]
    rx, ry, rz = (r_ij[:, 0].reshape(1, E), r_ij[:, 1].reshape(1, E),
                  r_ij[:, 2].reshape(1, E))
    Wd21 = jnp.concatenate([Wd, bd.reshape(1, H4)], axis=0)

    table = _build_table(s_j, vx, vy, vz, wx, wy, wz, W1, b1, W2, b2)
    RB = _rchain(rx, ry, rz).T
    i0r = i0.reshape(NROW, 1, RPB)

    ea = E // 2
    nchunks = ea // GCH
    small = nchunks // NT
    nbig = nchunks - small * NT

    def grp_idx(ig):
        split = nbig * (small + 1) * GCH
        a = ig[:split].reshape(nbig, (small + 1) * GCH)
        b = ig[split:].reshape(NT - nbig, small * GCH)
        b = jnp.concatenate(
            [b, jnp.zeros((NT - nbig, GCH), jnp.int32)], axis=1)
        return jnp.concatenate([a, b], axis=0).reshape(NT, small + 1, GCH)

    ga = _gather_rows(table, grp_idx(i1[:ea]), ea)
    gb = _gather_rows(table, grp_idx(i1[ea:]), ea)
    pa = _edge_stage(ga, RB[:ea], Wd21, ea)
    pb = _edge_stage(gb, RB[ea:], Wd21, ea)
    parts = _scatter_planes(pa, ga, pb, gb, i0r)
    A = parts.reshape(NC * NPL, N, FEAT)
    dh, dhbar, dvx, dvy, dvz, dbx, dby, dbz = _node_stage(
        A, s_j, sbar_j, vx, vy, vz, wx, wy, wz, Wdvbar, Wdv)
    dv = jnp.stack([dvx, dvy, dvz], axis=-1)
    dvbar = jnp.stack([dbx, dby, dbz], axis=-1)
    return dh, dhbar, dv, dvbar


# speedup vs baseline: 1.1541x; 1.0037x over previous
import functools

import jax
import jax.numpy as jnp
from jax import lax
from jax.experimental import pallas as pl
from jax.experimental.pallas import tpu as pltpu
from jax.experimental.pallas import tpu_sc as plsc

FEAT = 128
NRBF = 20
CUTOFF = 5.0
N = 10000
E = 160000
H4 = 4 * FEAT
TW = H4 + 6 * FEAT
PW = 7 * FEAT
NPL = 13

NC = 2
NS = 16
NT = NC * NS
EPT = E // NT
GCH = 40
NGC = EPT // GCH

RPB = 128
NROW = E // RPB
RPC = NROW // NC
RPT = RPC // NS
ZR = 208

_HI = lax.Precision.HIGHEST


def _prep_body(s_ref, vp_ref, wp_ref, W1_ref, b1_ref, W2_ref, b2_ref,
               rx_ref, ry_ref, rz_ref, t_ref, rb_ref):
    x = s_ref[...]
    pre = jnp.dot(x, W1_ref[...], preferred_element_type=jnp.float32,
                  precision=_HI) + b1_ref[...]
    act = pre * jax.nn.sigmoid(pre)
    h = jnp.dot(act, W2_ref[...], preferred_element_type=jnp.float32,
                precision=_HI) + b2_ref[...]
    t_ref[...] = jnp.concatenate(
        [h, vp_ref[0], vp_ref[1], vp_ref[2],
         wp_ref[0], wp_ref[1], wp_ref[2]], axis=1)

    rx = rx_ref[...]
    ry = ry_ref[...]
    rz = rz_ref[...]
    d2 = rx * rx + ry * ry + rz * rz + 3e-8
    inv = lax.rsqrt(d2)
    dist = d2 * inv
    a = (jnp.pi / CUTOFF) * dist
    s1 = jnp.sin(a)
    c1 = jnp.cos(a)
    env = jnp.where(dist < CUTOFF, 0.5 * (c1 + 1.0), 0.0)
    m = env * inv
    two_c = 2.0 * c1
    rows = [m * s1]
    sk_prev, sk = s1, two_c * s1
    rows.append(m * sk)
    for _ in range(2, NRBF):
        sk_prev, sk = sk, two_c * sk - sk_prev
        rows.append(m * sk)
    rows.append(env)
    rows.append(rx * inv)
    rows.append(ry * inv)
    rows.append(rz * inv)
    rb_ref[...] = jnp.concatenate(rows, axis=0)


def _prep_stage(s_j, vp, wp, W1, b1, W2, b2, rx, ry, rz):
    nb = 50
    bn = N // nb
    eb = E // nb
    row = lambda i: (i, 0)
    col = lambda i: (0, i)
    full = lambda shape: pl.BlockSpec(shape, lambda i: tuple([0] * len(shape)))
    return pl.pallas_call(
        _prep_body,
        grid=(nb,),
        in_specs=[
            pl.BlockSpec((bn, FEAT), row),
            pl.BlockSpec((3, bn, FEAT), lambda i: (0, i, 0)),
            pl.BlockSpec((3, bn, FEAT), lambda i: (0, i, 0)),
            full((FEAT, FEAT)), full((1, FEAT)),
            full((FEAT, H4)), full((1, H4)),
            pl.BlockSpec((1, eb), col),
            pl.BlockSpec((1, eb), col),
            pl.BlockSpec((1, eb), col),
        ],
        out_specs=[pl.BlockSpec((bn, TW), row),
                   pl.BlockSpec((NRBF + 4, eb), col)],
        out_shape=[jax.ShapeDtypeStruct((N, TW), jnp.float32),
                   jax.ShapeDtypeStruct((NRBF + 4, E), jnp.float32)],
    )(s_j, vp, wp, W1, b1.reshape(1, FEAT), W2, b2.reshape(1, H4), rx, ry, rz)


def _gather_rows(table, i1r, e_g):
    mesh = plsc.VectorSubcoreMesh(core_axis_name="c", subcore_axis_name="s")
    nchunks = e_g // GCH
    small = nchunks // NT
    nbig = nchunks - small * NT
    ngc = i1r.shape[1]
    assert ngc == small + (1 if nbig else 0)

    @functools.partial(
        pl.kernel,
        out_type=jax.ShapeDtypeStruct((e_g, TW), jnp.float32),
        mesh=mesh,
        scratch_types=[
            pltpu.VMEM((ngc, GCH), jnp.int32),
            pltpu.VMEM((GCH, TW), jnp.float32),
            pltpu.VMEM((GCH, TW), jnp.float32),
            pltpu.SemaphoreType.DMA,
            pltpu.SemaphoreType.DMA,
        ],
    )
    def k(t_hbm, idx_hbm, out_hbm, idx_v, buf_a, buf_b, sem_a, sem_b):
        w = lax.axis_index("c") * NS + lax.axis_index("s")
        pltpu.sync_copy(idx_hbm.at[w], idx_v)
        base = w * (small * GCH) + jnp.minimum(w, nbig) * GCH
        out_at = lambda kk: out_hbm.at[pl.ds(base + kk * GCH, GCH)]

        pltpu.async_copy(t_hbm.at[idx_v.at[0]], buf_a, sem_a)

        @pl.loop(0, small // 2)
        def _(i):
            kk = 2 * i
            pltpu.async_copy(t_hbm.at[idx_v.at[kk + 1]], buf_b, sem_b)
            pltpu.make_async_copy(t_hbm.at[idx_v.at[kk]], buf_a, sem_a).wait()
            pltpu.sync_copy(buf_a, out_at(kk))

            @pl.when(kk + 2 < small)
            def _():
                pltpu.async_copy(t_hbm.at[idx_v.at[kk + 2]], buf_a, sem_a)

            pltpu.make_async_copy(
                t_hbm.at[idx_v.at[kk + 1]], buf_b, sem_b).wait()
            pltpu.sync_copy(buf_b, out_at(kk + 1))

        if small % 2:
            pltpu.make_async_copy(
                t_hbm.at[idx_v.at[small - 1]], buf_a, sem_a).wait()
            pltpu.sync_copy(buf_a, out_at(small - 1))

        if nbig:
            @pl.when(w < nbig)
            def _():
                pltpu.async_copy(
                    t_hbm.at[idx_v.at[small]], buf_b, sem_b).wait()
                pltpu.sync_copy(buf_b, out_at(small))

    return k(table, i1r)


def _edge_body(g_ref, rb_ref, Wd_ref, p_ref):
    rb = rb_ref[...]
    ws = jnp.dot(rb[:, 0:NRBF + 1], Wd_ref[...],
                 preferred_element_type=jnp.float32, precision=_HI)
    ux = rb[:, NRBF + 1:NRBF + 2]
    uy = rb[:, NRBF + 2:NRBF + 3]
    uz = rb[:, NRBF + 3:NRBF + 4]
    g = g_ref[...]
    t0 = g[:, 0:FEAT] * ws[:, 0:FEAT]
    t1 = g[:, FEAT:2 * FEAT] * ws[:, FEAT:2 * FEAT]
    t2 = g[:, 2 * FEAT:3 * FEAT] * ws[:, 2 * FEAT:3 * FEAT]
    t3 = g[:, 3 * FEAT:4 * FEAT] * ws[:, 3 * FEAT:4 * FEAT]
    v1x = g[:, H4:H4 + FEAT]
    v1y = g[:, H4 + FEAT:H4 + 2 * FEAT]
    v1z = g[:, H4 + 2 * FEAT:H4 + 3 * FEAT]
    w1x = g[:, H4 + 3 * FEAT:H4 + 4 * FEAT]
    w1y = g[:, H4 + 4 * FEAT:H4 + 5 * FEAT]
    w1z = g[:, H4 + 5 * FEAT:H4 + 6 * FEAT]
    p_ref[...] = jnp.concatenate([
        t1,
        t2 * ux + t0 * v1x,
        t2 * uy + t0 * v1y,
        t2 * uz + t0 * v1z,
        t3 * w1x,
        t3 * w1y,
        t3 * w1z,
    ], axis=1)


def _edge_stage(G, RB, Wd21, e_g):
    eb = 640
    ng = e_g // eb
    row = lambda i: (i, 0)
    return pl.pallas_call(
        _edge_body,
        grid=(ng,),
        in_specs=[
            pl.BlockSpec((eb, TW), row),
            pl.BlockSpec((eb, NRBF + 4), row),
            pl.BlockSpec((NRBF + 1, H4), lambda i: (0, 0)),
        ],
        out_specs=pl.BlockSpec((eb, PW), row),
        out_shape=jax.ShapeDtypeStruct((e_g, PW), jnp.float32),
    )(G, RB, Wd21)


def _scatter_planes(PA, GA, PB, GB, roff_b, i0r):
    mesh = plsc.VectorSubcoreMesh(core_axis_name="c", subcore_axis_name="s")

    @functools.partial(
        pl.kernel,
        out_type=jax.ShapeDtypeStruct((NC, NPL, N, FEAT), jnp.float32),
        mesh=mesh,
        scratch_types=[
            pltpu.VMEM((RPT + 1, 1, RPB), jnp.int32),
            pltpu.VMEM((RPB, FEAT), jnp.float32),
            pltpu.VMEM((RPB, FEAT), jnp.float32),
            pltpu.VMEM_SHARED((N, FEAT), jnp.float32),
            pltpu.SemaphoreType.DMA,
            pltpu.SemaphoreType.DMA,
        ],
    )
    def k(pa_hbm, ga_hbm, pb_hbm, gb_hbm, idx_hbm, out_hbm,
          idx_v, pay_a, pay_b, acc, sem_a, sem_b):
        c = lax.axis_index("c")
        w = lax.axis_index("s")

        start = c * RPC + w * RPT
        pltpu.sync_copy(idx_hbm.at[pl.ds(start, RPT + 1)], idx_v)

        def zero_fill():
            @pl.loop(0, RPB)
            def _(i):
                @pl.loop(0, FEAT // 16)
                def _(l):
                    pay_a.at[i, pl.ds(l * 16, 16)][...] = jnp.zeros(
                        (16,), jnp.float32)

        zero_fill()

        for p in range(NPL):
            for z in range(4):
                pltpu.sync_copy(pay_a,
                                acc.at[pl.ds(w * 624 + z * RPB, RPB)])
            pltpu.sync_copy(pay_a.at[pl.ds(0, 112)],
                            acc.at[pl.ds(w * 624 + 4 * RPB, 112)])

            @pl.when(w == NS - 1)
            def _():
                pltpu.sync_copy(pay_a.at[pl.ds(0, 16)],
                                acc.at[pl.ds(N - 16, 16)])

            plsc.subcore_barrier()

            def emit(p_src, g_src, roff):
                if p < 7:
                    src, col = p_src, p * FEAT
                else:
                    src, col = g_src, H4 + (p - 7) * FEAT

                def src_at(j):
                    return src.at[pl.ds((roff + w * RPT + j) * RPB, RPB),
                                  pl.ds(col, FEAT)]

                def fetch(j, buf, sem):
                    pltpu.async_copy(src_at(j), buf, sem)

                def drain_add(j, buf, sem):
                    pltpu.make_async_copy(src_at(j), buf, sem).wait()
                    pltpu.sync_copy(buf, acc.at[idx_v.at[j, 0]], add=True)

                fetch(0, pay_b, sem_b)

                @pl.loop(0, RPT // 2)
                def _(i):
                    j = 2 * i
                    fetch(j + 1, pay_a, sem_a)
                    drain_add(j, pay_b, sem_b)

                    @pl.when(j + 2 < RPT)
                    def _():
                        fetch(j + 2, pay_b, sem_b)

                    drain_add(j + 1, pay_a, sem_a)

                if RPT % 2:
                    drain_add(RPT - 1, pay_b, sem_b)

                @pl.when(w == NS - 1)
                def _():
                    pltpu.sync_copy(src_at(RPT), pay_b)
                    pltpu.sync_copy(pay_b, acc.at[idx_v.at[RPT, 0]],
                                    add=True)

            @pl.when(c == 0)
            def _():
                emit(pa_hbm, ga_hbm, 0)

            @pl.when(c == 1)
            def _():
                emit(pb_hbm, gb_hbm, roff_b)

            plsc.subcore_barrier()
            pltpu.sync_copy(acc.at[pl.ds(w * 624, 624)],
                            out_hbm.at[c, p, pl.ds(w * 624, 624)])

            @pl.when(w == NS - 1)
            def _():
                pltpu.sync_copy(acc.at[pl.ds(N - 16, 16)],
                                out_hbm.at[c, p, pl.ds(N - 16, 16)])

            if p + 1 < NPL:
                zero_fill()

    return k(PA, GA, PB, GB, i0r)


def _node_body(a_ref, s_ref, sb_ref, vp_ref, wp_ref,
               Wvb_ref, Wv_ref,
               dh_ref, dhb_ref, dvx_r, dvy_r, dvz_r, dbx_r, dby_r, dbz_r):
    agg = lambda p: a_ref[p] + a_ref[NPL + p]
    dh = agg(0)
    pv = (agg(1), agg(2), agg(3))
    B = (agg(4), agg(5), agg(6))
    Av = (agg(7), agg(8), agg(9))
    Aw = (agg(10), agg(11), agg(12))
    s = s_ref[...]
    sb = sb_ref[...]
    v = (vp_ref[0], vp_ref[1], vp_ref[2])
    vb = (wp_ref[0], wp_ref[1], wp_ref[2])
    dh_ref[...] = dh
    dhb_ref[...] = v[0] * B[0] + v[1] * B[1] + v[2] * B[2]

    def cross(a, b, d):
        i, j = (d + 1) % 3, (d + 2) % 3
        return a[i] * b[j] - a[j] * b[i]

    Wvb = Wvb_ref[...]
    Wv = Wv_ref[...]
    douts = (dvx_r, dvy_r, dvz_r)
    bouts = (dbx_r, dby_r, dbz_r)
    for d in range(3):
        catb = jnp.concatenate(
            [s * Aw[d], sb * Av[d], cross(v, Av, d), cross(vb, Aw, d)], axis=1)
        bouts[d][...] = jnp.dot(catb, Wvb, preferred_element_type=jnp.float32,
                                precision=_HI)
        catv = jnp.concatenate(
            [s * Av[d], sb * Aw[d], cross(v, Aw, d)], axis=1)
        douts[d][...] = pv[d] + jnp.dot(
            catv, Wv, preferred_element_type=jnp.float32, precision=_HI)


def _node_stage(A, s_j, sbar_j, vp, wp, Wdvbar, Wdv):
    nb = 10
    bn = N // nb
    row = lambda i: (i, 0)
    o = pl.BlockSpec((bn, FEAT), row)
    outs = [jax.ShapeDtypeStruct((N, FEAT), jnp.float32)] * 8
    return pl.pallas_call(
        _node_body,
        grid=(nb,),
        in_specs=[
            pl.BlockSpec((2 * NPL, bn, FEAT), lambda i: (0, i, 0)),
            pl.BlockSpec((bn, FEAT), row),
            pl.BlockSpec((bn, FEAT), row),
            pl.BlockSpec((3, bn, FEAT), lambda i: (0, i, 0)),
            pl.BlockSpec((3, bn, FEAT), lambda i: (0, i, 0)),
            pl.BlockSpec((H4, FEAT), lambda i: (0, 0)),
            pl.BlockSpec((3 * FEAT, FEAT), lambda i: (0, 0)),
        ],
        out_specs=[o] * 8,
        out_shape=outs,
    )(A, s_j, sbar_j, vp, wp, Wdvbar, Wdv)


def kernel(s_j, sbar_j, v_j, vbar_j, r_ij, nbrs, W1, b1, W2, b2, Wd, bd,
           Wdvbar, Wdv):
    i0 = nbrs[:, 0].astype(jnp.int32)
    i1 = nbrs[:, 1].astype(jnp.int32)
    vp = jnp.moveaxis(v_j, 2, 0)
    wp = jnp.moveaxis(vbar_j, 2, 0)
    rt = r_ij.T.reshape(3, 1, E)
    Wd21 = jnp.concatenate([Wd, bd.reshape(1, H4)], axis=0)

    table, RBt = _prep_stage(s_j, vp, wp, W1, b1, W2, b2,
                             rt[0], rt[1], rt[2])
    RB = RBt.T
    i0r = i0.reshape(NROW, 1, RPB)

    G = _gather_rows(table, i1.reshape(NT, E // (NT * GCH), GCH), E)
    P = _edge_stage(G, RB, Wd21, E)
    parts = _scatter_planes(P, G, P, G, RPC, i0r)
    A = parts.reshape(NC * NPL, N, FEAT)
    dh, dhbar, dvx, dvy, dvz, dbx, dby, dbz = _node_stage(
        A, s_j, sbar_j, vp, wp, Wdvbar, Wdv)
    dv = jnp.stack([dvx, dvy, dvz], axis=-1)
    dvbar = jnp.stack([dbx, dby, dbz], axis=-1)
    return dh, dhbar, dv, dvbar


# speedup vs baseline: 1.1571x; 1.0026x over previous
import functools

import jax
import jax.numpy as jnp
from jax import lax
from jax.experimental import pallas as pl
from jax.experimental.pallas import tpu as pltpu
from jax.experimental.pallas import tpu_sc as plsc

FEAT = 128
NRBF = 20
CUTOFF = 5.0
N = 10000
E = 160000
H4 = 4 * FEAT
TW = H4 + 6 * FEAT
PW = 7 * FEAT
NPL = 13

NC = 2
NS = 16
NT = NC * NS
EPT = E // NT
GCH = 40

RPB = 128
NROW = E // RPB
RPC = NROW // NC
RPT = RPC // NS
ZR = 48

_HI = lax.Precision.HIGHEST


def _prep_body(s_ref, vp_ref, wp_ref, W1_ref, b1_ref, W2_ref, b2_ref,
               rx_ref, ry_ref, rz_ref, t_ref, rb_ref):
    x = s_ref[...]
    pre = jnp.dot(x, W1_ref[...], preferred_element_type=jnp.float32,
                  precision=_HI) + b1_ref[...]
    act = pre * jax.nn.sigmoid(pre)
    h = jnp.dot(act, W2_ref[...], preferred_element_type=jnp.float32,
                precision=_HI) + b2_ref[...]
    t_ref[...] = jnp.concatenate(
        [h, vp_ref[0], vp_ref[1], vp_ref[2],
         wp_ref[0], wp_ref[1], wp_ref[2]], axis=1)

    rx = rx_ref[...]
    ry = ry_ref[...]
    rz = rz_ref[...]
    d2 = rx * rx + ry * ry + rz * rz + 3e-8
    inv = lax.rsqrt(d2)
    dist = d2 * inv
    a = (jnp.pi / CUTOFF) * dist
    s1 = jnp.sin(a)
    c1 = jnp.cos(a)
    env = jnp.where(dist < CUTOFF, 0.5 * (c1 + 1.0), 0.0)
    m = env * inv
    two_c = 2.0 * c1
    rows = [m * s1]
    sk_prev, sk = s1, two_c * s1
    rows.append(m * sk)
    for _ in range(2, NRBF):
        sk_prev, sk = sk, two_c * sk - sk_prev
        rows.append(m * sk)
    rows.append(env)
    rows.append(rx * inv)
    rows.append(ry * inv)
    rows.append(rz * inv)
    rb_ref[...] = jnp.concatenate(rows, axis=0)


def _prep_stage(s_j, vp, wp, W1, b1, W2, b2, rx, ry, rz):
    nb = 50
    bn = N // nb
    eb = E // nb
    row = lambda i: (i, 0)
    col = lambda i: (0, i)
    full = lambda shape: pl.BlockSpec(shape, lambda i: tuple([0] * len(shape)))
    return pl.pallas_call(
        _prep_body,
        grid=(nb,),
        in_specs=[
            pl.BlockSpec((bn, FEAT), row),
            pl.BlockSpec((3, bn, FEAT), lambda i: (0, i, 0)),
            pl.BlockSpec((3, bn, FEAT), lambda i: (0, i, 0)),
            full((FEAT, FEAT)), full((1, FEAT)),
            full((FEAT, H4)), full((1, H4)),
            pl.BlockSpec((1, eb), col),
            pl.BlockSpec((1, eb), col),
            pl.BlockSpec((1, eb), col),
        ],
        out_specs=[pl.BlockSpec((bn, TW), row),
                   pl.BlockSpec((NRBF + 4, eb), col)],
        out_shape=[jax.ShapeDtypeStruct((N, TW), jnp.float32),
                   jax.ShapeDtypeStruct((NRBF + 4, E), jnp.float32)],
    )(s_j, vp, wp, W1, b1.reshape(1, FEAT), W2, b2.reshape(1, H4), rx, ry, rz)


def _gather_rows(table, i1r, e_g):
    mesh = plsc.VectorSubcoreMesh(core_axis_name="c", subcore_axis_name="s")
    ept = e_g // NT
    nfull = ept // GCH
    tail = ept - nfull * GCH
    assert tail % 8 == 0 and i1r.shape[1] == nfull + (1 if tail else 0)
    ngc = i1r.shape[1]

    @functools.partial(
        pl.kernel,
        out_type=jax.ShapeDtypeStruct((e_g, TW), jnp.float32),
        mesh=mesh,
        scratch_types=[
            pltpu.VMEM((ngc, GCH), jnp.int32),
            pltpu.VMEM((GCH, TW), jnp.float32),
            pltpu.VMEM((GCH, TW), jnp.float32),
            pltpu.SemaphoreType.DMA,
            pltpu.SemaphoreType.DMA,
        ],
    )
    def k(t_hbm, idx_hbm, out_hbm, idx_v, buf_a, buf_b, sem_a, sem_b):
        w = lax.axis_index("c") * NS + lax.axis_index("s")
        pltpu.sync_copy(idx_hbm.at[w], idx_v)
        base = w * ept
        out_at = lambda kk: out_hbm.at[pl.ds(base + kk * GCH, GCH)]

        pltpu.async_copy(t_hbm.at[idx_v.at[0]], buf_a, sem_a)

        @pl.loop(0, nfull // 2)
        def _(i):
            kk = 2 * i
            pltpu.async_copy(t_hbm.at[idx_v.at[kk + 1]], buf_b, sem_b)
            pltpu.make_async_copy(t_hbm.at[idx_v.at[kk]], buf_a, sem_a).wait()
            pltpu.sync_copy(buf_a, out_at(kk))

            @pl.when(kk + 2 < nfull)
            def _():
                pltpu.async_copy(t_hbm.at[idx_v.at[kk + 2]], buf_a, sem_a)

            pltpu.make_async_copy(
                t_hbm.at[idx_v.at[kk + 1]], buf_b, sem_b).wait()
            pltpu.sync_copy(buf_b, out_at(kk + 1))

        if nfull % 2:
            pltpu.make_async_copy(
                t_hbm.at[idx_v.at[nfull - 1]], buf_a, sem_a).wait()
            pltpu.sync_copy(buf_a, out_at(nfull - 1))

        if tail:
            pltpu.async_copy(t_hbm.at[idx_v.at[nfull]], buf_b, sem_b).wait()
            pltpu.sync_copy(buf_b.at[pl.ds(0, tail)],
                            out_hbm.at[pl.ds(base + nfull * GCH, tail)])

    return k(table, i1r)


def _edge_body(g_ref, rb_ref, Wd_ref, p_ref):
    rb = rb_ref[...]
    ws = jnp.dot(rb[:, 0:NRBF + 1], Wd_ref[...],
                 preferred_element_type=jnp.float32, precision=_HI)
    ux = rb[:, NRBF + 1:NRBF + 2]
    uy = rb[:, NRBF + 2:NRBF + 3]
    uz = rb[:, NRBF + 3:NRBF + 4]
    g = g_ref[...]
    t0 = g[:, 0:FEAT] * ws[:, 0:FEAT]
    t1 = g[:, FEAT:2 * FEAT] * ws[:, FEAT:2 * FEAT]
    t2 = g[:, 2 * FEAT:3 * FEAT] * ws[:, 2 * FEAT:3 * FEAT]
    t3 = g[:, 3 * FEAT:4 * FEAT] * ws[:, 3 * FEAT:4 * FEAT]
    v1x = g[:, H4:H4 + FEAT]
    v1y = g[:, H4 + FEAT:H4 + 2 * FEAT]
    v1z = g[:, H4 + 2 * FEAT:H4 + 3 * FEAT]
    w1x = g[:, H4 + 3 * FEAT:H4 + 4 * FEAT]
    w1y = g[:, H4 + 4 * FEAT:H4 + 5 * FEAT]
    w1z = g[:, H4 + 5 * FEAT:H4 + 6 * FEAT]
    p_ref[...] = jnp.concatenate([
        t1,
        t2 * ux + t0 * v1x,
        t2 * uy + t0 * v1y,
        t2 * uz + t0 * v1z,
        t3 * w1x,
        t3 * w1y,
        t3 * w1z,
    ], axis=1)


def _edge_stage(G, RB, Wd21, e_g):
    eb = 640
    ng = e_g // eb
    row = lambda i: (i, 0)
    return pl.pallas_call(
        _edge_body,
        grid=(ng,),
        in_specs=[
            pl.BlockSpec((eb, TW), row),
            pl.BlockSpec((eb, NRBF + 4), row),
            pl.BlockSpec((NRBF + 1, H4), lambda i: (0, 0)),
        ],
        out_specs=pl.BlockSpec((eb, PW), row),
        out_shape=jax.ShapeDtypeStruct((e_g, PW), jnp.float32),
    )(G, RB, Wd21)


def _scatter_planes(PA, GA, PB, GB, roff_b, i0r):
    mesh = plsc.VectorSubcoreMesh(core_axis_name="c", subcore_axis_name="s")

    @functools.partial(
        pl.kernel,
        out_type=jax.ShapeDtypeStruct((NC, NPL, N, FEAT), jnp.float32),
        mesh=mesh,
        scratch_types=[
            pltpu.VMEM((RPT + 1, 1, RPB), jnp.int32),
            pltpu.VMEM((RPB, FEAT), jnp.float32),
            pltpu.VMEM((RPB, FEAT), jnp.float32),
            pltpu.VMEM((ZR, FEAT), jnp.float32),
            pltpu.VMEM_SHARED((N, FEAT), jnp.float32),
            pltpu.SemaphoreType.DMA,
            pltpu.SemaphoreType.DMA,
        ],
    )
    def k(pa_hbm, ga_hbm, pb_hbm, gb_hbm, idx_hbm, out_hbm,
          idx_v, pay_a, pay_b, zero_v, acc, sem_a, sem_b):
        c = lax.axis_index("c")
        w = lax.axis_index("s")

        start = c * RPC + w * RPT
        pltpu.sync_copy(idx_hbm.at[pl.ds(start, RPT + 1)], idx_v)

        @pl.loop(0, ZR)
        def _(i):
            @pl.loop(0, FEAT // 16)
            def _(l):
                zero_v.at[i, pl.ds(l * 16, 16)][...] = jnp.zeros(
                    (16,), jnp.float32)

        for p in range(NPL):
            for z in range(624 // ZR):
                pltpu.sync_copy(zero_v,
                                acc.at[pl.ds(w * 624 + z * ZR, ZR)])

            @pl.when(w == NS - 1)
            def _():
                pltpu.sync_copy(zero_v.at[pl.ds(0, 16)],
                                acc.at[pl.ds(N - 16, 16)])

            plsc.subcore_barrier()

            def emit(p_src, g_src, roff):
                if p < 7:
                    src, col = p_src, p * FEAT
                else:
                    src, col = g_src, H4 + (p - 7) * FEAT

                def src_at(j):
                    return src.at[pl.ds((roff + w * RPT + j) * RPB, RPB),
                                  pl.ds(col, FEAT)]

                def fetch(j, buf, sem):
                    pltpu.async_copy(src_at(j), buf, sem)

                def drain_add(j, buf, sem):
                    pltpu.make_async_copy(src_at(j), buf, sem).wait()
                    pltpu.sync_copy(buf, acc.at[idx_v.at[j, 0]], add=True)

                fetch(0, pay_b, sem_b)

                @pl.loop(0, RPT // 2)
                def _(i):
                    j = 2 * i
                    fetch(j + 1, pay_a, sem_a)
                    drain_add(j, pay_b, sem_b)

                    @pl.when(j + 2 < RPT)
                    def _():
                        fetch(j + 2, pay_b, sem_b)

                    drain_add(j + 1, pay_a, sem_a)

                if RPT % 2:
                    drain_add(RPT - 1, pay_b, sem_b)

                @pl.when(w == NS - 1)
                def _():
                    pltpu.sync_copy(src_at(RPT), pay_b)
                    pltpu.sync_copy(pay_b, acc.at[idx_v.at[RPT, 0]],
                                    add=True)

            @pl.when(c == 0)
            def _():
                emit(pa_hbm, ga_hbm, 0)

            @pl.when(c == 1)
            def _():
                emit(pb_hbm, gb_hbm, roff_b)

            plsc.subcore_barrier()
            pltpu.sync_copy(acc.at[pl.ds(w * 624, 624)],
                            out_hbm.at[c, p, pl.ds(w * 624, 624)])

            @pl.when(w == NS - 1)
            def _():
                pltpu.sync_copy(acc.at[pl.ds(N - 16, 16)],
                                out_hbm.at[c, p, pl.ds(N - 16, 16)])

    return k(PA, GA, PB, GB, i0r)


def _node_body(a_ref, s_ref, sb_ref, vp_ref, wp_ref,
               Wvb_ref, Wv_ref,
               dh_ref, dhb_ref, dvx_r, dvy_r, dvz_r, dbx_r, dby_r, dbz_r):
    agg = lambda p: a_ref[p] + a_ref[NPL + p]
    dh = agg(0)
    pv = (agg(1), agg(2), agg(3))
    B = (agg(4), agg(5), agg(6))
    Av = (agg(7), agg(8), agg(9))
    Aw = (agg(10), agg(11), agg(12))
    s = s_ref[...]
    sb = sb_ref[...]
    v = (vp_ref[0], vp_ref[1], vp_ref[2])
    vb = (wp_ref[0], wp_ref[1], wp_ref[2])
    dh_ref[...] = dh
    dhb_ref[...] = v[0] * B[0] + v[1] * B[1] + v[2] * B[2]

    def cross(a, b, d):
        i, j = (d + 1) % 3, (d + 2) % 3
        return a[i] * b[j] - a[j] * b[i]

    Wvb = Wvb_ref[...]
    Wv = Wv_ref[...]
    douts = (dvx_r, dvy_r, dvz_r)
    bouts = (dbx_r, dby_r, dbz_r)
    for d in range(3):
        catb = jnp.concatenate(
            [s * Aw[d], sb * Av[d], cross(v, Av, d), cross(vb, Aw, d)], axis=1)
        bouts[d][...] = jnp.dot(catb, Wvb, preferred_element_type=jnp.float32,
                                precision=_HI)
        catv = jnp.concatenate(
            [s * Av[d], sb * Aw[d], cross(v, Aw, d)], axis=1)
        douts[d][...] = pv[d] + jnp.dot(
            catv, Wv, preferred_element_type=jnp.float32, precision=_HI)


def _node_stage(A, s_j, sbar_j, vp, wp, Wdvbar, Wdv):
    nb = 10
    bn = N // nb
    row = lambda i: (i, 0)
    o = pl.BlockSpec((bn, FEAT), row)
    outs = [jax.ShapeDtypeStruct((N, FEAT), jnp.float32)] * 8
    return pl.pallas_call(
        _node_body,
        grid=(nb,),
        in_specs=[
            pl.BlockSpec((2 * NPL, bn, FEAT), lambda i: (0, i, 0)),
            pl.BlockSpec((bn, FEAT), row),
            pl.BlockSpec((bn, FEAT), row),
            pl.BlockSpec((3, bn, FEAT), lambda i: (0, i, 0)),
            pl.BlockSpec((3, bn, FEAT), lambda i: (0, i, 0)),
            pl.BlockSpec((H4, FEAT), lambda i: (0, 0)),
            pl.BlockSpec((3 * FEAT, FEAT), lambda i: (0, 0)),
        ],
        out_specs=[o] * 8,
        out_shape=outs,
    )(A, s_j, sbar_j, vp, wp, Wdvbar, Wdv)


def kernel(s_j, sbar_j, v_j, vbar_j, r_ij, nbrs, W1, b1, W2, b2, Wd, bd,
           Wdvbar, Wdv):
    i0 = nbrs[:, 0].astype(jnp.int32)
    i1 = nbrs[:, 1].astype(jnp.int32)
    vp = jnp.moveaxis(v_j, 2, 0)
    wp = jnp.moveaxis(vbar_j, 2, 0)
    rt = r_ij.T.reshape(3, 1, E)
    Wd21 = jnp.concatenate([Wd, bd.reshape(1, H4)], axis=0)

    table, RBt = _prep_stage(s_j, vp, wp, W1, b1, W2, b2,
                             rt[0], rt[1], rt[2])
    RB = RBt.T
    i0r = i0.reshape(NROW, 1, RPB)

    ept = E // NT
    ngc = -(-ept // GCH)
    i1p = jnp.concatenate(
        [i1.reshape(NT, ept),
         jnp.zeros((NT, ngc * GCH - ept), jnp.int32)], axis=1)
    G = _gather_rows(table, i1p.reshape(NT, ngc, GCH), E)
    P = _edge_stage(G, RB, Wd21, E)
    parts = _scatter_planes(P, G, P, G, RPC, i0r)
    A = parts.reshape(NC * NPL, N, FEAT)
    dh, dhbar, dvx, dvy, dvz, dbx, dby, dbz = _node_stage(
        A, s_j, sbar_j, vp, wp, Wdvbar, Wdv)
    dv = jnp.stack([dvx, dvy, dvz], axis=-1)
    dvbar = jnp.stack([dbx, dby, dbz], axis=-1)
    return dh, dhbar, dv, dvbar
